# baseline (speedup 1.0000x reference)
import functools

import jax
import jax.numpy as jnp
from jax.experimental import pallas as pl
from jax.experimental.pallas import tpu as pltpu

_NB = (10, 20, 100)
_MAXK = 100
_T = 0.07
_NCLS = 1000

_QT = 256
_KT = 2048


def _matmul_body(nk_valid, q_ref, t_ref, o_ref):
    kidx = pl.program_id(1)
    sim = jax.lax.dot_general(
        q_ref[...], t_ref[...], (((1,), (1,)), ((), ())),
        preferred_element_type=jnp.float32)
    limit = nk_valid - kidx * _KT
    col = jax.lax.broadcasted_iota(jnp.int32, sim.shape, 1)
    o_ref[...] = jnp.where(col < limit, sim, -1e30)


def _similarity(features_rank, train_features_padded, nk_valid):
    q, _ = features_rank.shape
    kpad, _ = train_features_padded.shape
    grid = (q // _QT, kpad // _KT)
    return pl.pallas_call(
        functools.partial(_matmul_body, nk_valid),
        grid=grid,
        in_specs=[
            pl.BlockSpec((_QT, 128), lambda i, j: (i, 0)),
            pl.BlockSpec((_KT, 128), lambda i, j: (j, 0)),
        ],
        out_specs=pl.BlockSpec((_QT, _KT), lambda i, j: (i, j)),
        out_shape=jax.ShapeDtypeStruct((q, kpad), jnp.float32),
        compiler_params=pltpu.CompilerParams(
            dimension_semantics=("parallel", "parallel")),
    )(features_rank, train_features_padded)


def _vote_body(s_ref, l_ref, o10_ref, o20_ref, o100_ref):
    s = s_ref[...]
    lbl = l_ref[...]
    m = jnp.max(s, axis=1, keepdims=True)
    e = jnp.exp((s - m) / _T)
    w = e / jnp.sum(e, axis=1, keepdims=True)
    cls = jax.lax.broadcasted_iota(jnp.int32, (s.shape[0], _NCLS), 1)
    acc = jnp.zeros((s.shape[0], _NCLS), jnp.float32)

    def add_slot(acc, j):
        lj = lbl[:, j:j + 1]
        wj = w[:, j:j + 1]
        return acc + jnp.where(lj == cls, wj, 0.0)

    for j in range(10):
        acc = add_slot(acc, j)
    o10_ref[...] = acc
    for j in range(10, 20):
        acc = add_slot(acc, j)
    o20_ref[...] = acc
    for j in range(20, 100):
        acc = add_slot(acc, j)
    o100_ref[...] = acc


def _vote(topk_sims_padded, labels_padded):
    q = topk_sims_padded.shape[0]
    grid = (q // _QT,)
    out_sds = jax.ShapeDtypeStruct((q, _NCLS), jnp.float32)
    return pl.pallas_call(
        _vote_body,
        grid=grid,
        in_specs=[
            pl.BlockSpec((_QT, 128), lambda i: (i, 0)),
            pl.BlockSpec((_QT, 128), lambda i: (i, 0)),
        ],
        out_specs=[
            pl.BlockSpec((_QT, _NCLS), lambda i: (i, 0)),
            pl.BlockSpec((_QT, _NCLS), lambda i: (i, 0)),
            pl.BlockSpec((_QT, _NCLS), lambda i: (i, 0)),
        ],
        out_shape=[out_sds, out_sds, out_sds],
        compiler_params=pltpu.CompilerParams(
            dimension_semantics=("parallel",)),
    )(topk_sims_padded, labels_padded)


def kernel(train_features, train_labels, features_rank):
    nk = train_features.shape[0]
    kpad = ((nk + _KT - 1) // _KT) * _KT
    tf = jnp.pad(train_features, ((0, kpad - nk), (0, 0)))
    sim = _similarity(features_rank, tf, nk)
    topk_sims, indices = jax.lax.top_k(sim, _MAXK)
    labels = jnp.take(train_labels, indices, axis=0, mode="clip")
    sims_p = jnp.pad(topk_sims, ((0, 0), (0, 128 - _MAXK)),
                     constant_values=-1e30)
    labels_p = jnp.pad(labels, ((0, 0), (0, 128 - _MAXK)))
    o10, o20, o100 = _vote(sims_p, labels_p)
    return (o10, o20, o100)


# speedup vs baseline: 14.3770x; 14.3770x over previous
import functools

import jax
import jax.numpy as jnp
from jax import lax
from jax.experimental import pallas as pl
from jax.experimental.pallas import tpu as pltpu

_MAXK = 100
_T = 0.07
_NCLS = 1000

_QT = 256
_KT = 2048
_CH = 64
_NSEL = 128
_INT_MIN = -2147483648


def _k1_body(nk_valid, q_ref, t_ref, sim_ref, mx_ref):
    kidx = pl.program_id(1)
    sim = lax.dot_general(q_ref[...], t_ref[...], (((1,), (1,)), ((), ())),
                          preferred_element_type=jnp.float32)
    limit = nk_valid - kidx * _KT
    col = lax.broadcasted_iota(jnp.int32, sim.shape, 1)
    sim = jnp.where(col < limit, sim, -1e30)
    sim_ref[...] = sim
    nch = _KT // _CH
    cmx = jnp.max(sim.reshape(sim.shape[0], nch, _CH), axis=2)
    @pl.when(kidx % 4 == 0)
    def _():
        mx_ref[...] = jnp.full_like(mx_ref[...], -3e38)
    off = (kidx % 4) * nch
    lane_i = lax.broadcasted_iota(jnp.int32, (sim.shape[0], 128), 1)
    full = jnp.concatenate([cmx] * 4, axis=1)
    keep = (lane_i >= off) & (lane_i < off + nch)
    mx_ref[...] = jnp.where(keep, full, mx_ref[...])


def _k1(features_rank, tf_pad, nk_valid):
    q = features_rank.shape[0]
    kpad = tf_pad.shape[0]
    grid = (q // _QT, kpad // _KT)
    return pl.pallas_call(
        functools.partial(_k1_body, nk_valid),
        grid=grid,
        in_specs=[
            pl.BlockSpec((_QT, 128), lambda i, j: (i, 0)),
            pl.BlockSpec((_KT, 128), lambda i, j: (j, 0)),
        ],
        out_specs=[
            pl.BlockSpec((_QT, _KT), lambda i, j: (i, j)),
            pl.BlockSpec((_QT, 128), lambda i, j: (i, j // 4)),
        ],
        out_shape=[
            jax.ShapeDtypeStruct((q, kpad), jnp.float32),
            jax.ShapeDtypeStruct((q, kpad // _CH), jnp.float32),
        ],
        compiler_params=pltpu.CompilerParams(
            dimension_semantics=("parallel", "arbitrary")),
    )(features_rank, tf_pad)


def _k2_body(m_ref, tau_ref):
    b = lax.bitcast_convert_type(m_ref[...], jnp.int32)
    keys = jnp.where(b >= 0, b, (~b) ^ jnp.int32(_INT_MIN))
    n = keys.shape[0]
    lo = jnp.full((n, 1), _INT_MIN, jnp.int32)
    hi = jnp.full((n, 1), jnp.int32(2147483647), jnp.int32)
    for _ in range(32):
        mid = (lo & hi) + ((lo ^ hi) >> 1)
        mid = mid + ((lo ^ hi) & 1)
        cnt = jnp.sum((keys >= mid).astype(jnp.int32), axis=1, keepdims=True)
        ok = cnt >= _MAXK
        lo = jnp.where(ok, mid, lo)
        hi = jnp.where(ok, hi, mid - 1)
    bb = jnp.where(lo >= 0, lo, ~(lo ^ jnp.int32(_INT_MIN)))
    tau_ref[...] = lax.bitcast_convert_type(bb, jnp.float32)


def _k2(mx):
    q, c = mx.shape
    return pl.pallas_call(
        _k2_body,
        grid=(q // _QT,),
        in_specs=[pl.BlockSpec((_QT, c), lambda i: (i, 0))],
        out_specs=pl.BlockSpec((_QT, 1), lambda i: (i, 0)),
        out_shape=jax.ShapeDtypeStruct((q, 1), jnp.float32),
        compiler_params=pltpu.CompilerParams(
            dimension_semantics=("parallel",)),
    )(mx)


def _k3_body(v_ref, l_ref, t_ref, o10_ref, o20_ref, o100_ref):
    v_in = v_ref[...]
    lab = l_ref[...]
    tau = t_ref[...]
    n = v_in.shape[0]
    v0 = jnp.where(v_in >= tau, v_in, -3e38)
    lane = lax.broadcasted_iota(jnp.int32, v0.shape, 1)
    cls = lax.broadcasted_iota(jnp.int32, (n, _NCLS), 1)
    m0 = jnp.max(v0, axis=1, keepdims=True)

    def step(j, carry):
        v, acc, z = carry
        mj = jnp.max(v, axis=1, keepdims=True)
        ej = jnp.exp((mj - m0) / _T)
        pos = jnp.min(jnp.where(v == mj, lane, jnp.int32(2**30)),
                      axis=1, keepdims=True)
        labj = jnp.sum(jnp.where(lane == pos, lab, 0), axis=1, keepdims=True)
        acc = acc + jnp.where(cls == labj, ej, 0.0)
        z = z + ej
        v = jnp.where(lane == pos, -3e38, v)

        @pl.when(j == 9)
        def _():
            o10_ref[...] = acc

        @pl.when(j == 19)
        def _():
            o20_ref[...] = acc

        return v, acc, z

    _, acc, z = lax.fori_loop(
        0, _MAXK, step,
        (v0, jnp.zeros((n, _NCLS), jnp.float32), jnp.zeros((n, 1), jnp.float32)))
    o100_ref[...] = acc / z
    o10_ref[...] = o10_ref[...] / z
    o20_ref[...] = o20_ref[...] / z


def _k3(cval, clab, tau):
    q, w = cval.shape
    qt = 64
    out_sds = jax.ShapeDtypeStruct((q, _NCLS), jnp.float32)
    return pl.pallas_call(
        _k3_body,
        grid=(q // qt,),
        in_specs=[
            pl.BlockSpec((qt, w), lambda i: (i, 0)),
            pl.BlockSpec((qt, w), lambda i: (i, 0)),
            pl.BlockSpec((qt, 1), lambda i: (i, 0)),
        ],
        out_specs=[
            pl.BlockSpec((qt, _NCLS), lambda i: (i, 0)),
            pl.BlockSpec((qt, _NCLS), lambda i: (i, 0)),
            pl.BlockSpec((qt, _NCLS), lambda i: (i, 0)),
        ],
        out_shape=[out_sds, out_sds, out_sds],
        compiler_params=pltpu.CompilerParams(
            dimension_semantics=("arbitrary",)),
    )(cval, clab, tau)


def kernel(train_features, train_labels, features_rank):
    nk = train_features.shape[0]
    blk = _KT * 4
    kpad = ((nk + blk - 1) // blk) * blk
    nchunks = kpad // _CH
    tf = jnp.pad(train_features, ((0, kpad - nk), (0, 0)))
    sim, mx = _k1(features_rank, tf, nk)
    tau = _k2(mx)
    _, cidx = lax.top_k(mx, _NSEL)
    sim3 = sim.reshape(-1, nchunks, _CH)
    cval = jnp.take_along_axis(sim3, cidx[:, :, None], axis=1)
    labrows = jnp.pad(train_labels, (0, kpad - nk)).reshape(nchunks, _CH)
    clab = jnp.take(labrows, cidx, axis=0)
    q = features_rank.shape[0]
    o10, o20, o100 = _k3(cval.reshape(q, -1), clab.reshape(q, -1), tau)
    return (o10, o20, o100)


# speedup vs baseline: 14.4513x; 1.0052x over previous
import functools

import jax
import jax.numpy as jnp
from jax import lax
from jax.experimental import pallas as pl
from jax.experimental.pallas import tpu as pltpu

_MAXK = 100
_T = 0.07
_NCLS = 1000

_QT = 256
_KT = 2048
_CH = 64
_NSEL = 128
_INT_MIN = -2147483648


def _k1_body(nk_valid, q_ref, t_ref, sim_ref, mx_ref):
    kidx = pl.program_id(1)
    sim = lax.dot_general(q_ref[...], t_ref[...], (((1,), (1,)), ((), ())),
                          preferred_element_type=jnp.float32)
    limit = nk_valid - kidx * _KT
    col = lax.broadcasted_iota(jnp.int32, sim.shape, 1)
    sim = jnp.where(col < limit, sim, -1e30)
    sim_ref[...] = sim
    nch = _KT // _CH
    cmx = jnp.max(sim.reshape(sim.shape[0], nch, _CH), axis=2)
    @pl.when(kidx % 4 == 0)
    def _():
        mx_ref[...] = jnp.full_like(mx_ref[...], -3e38)
    off = (kidx % 4) * nch
    lane_i = lax.broadcasted_iota(jnp.int32, (sim.shape[0], 128), 1)
    full = jnp.concatenate([cmx] * 4, axis=1)
    keep = (lane_i >= off) & (lane_i < off + nch)
    mx_ref[...] = jnp.where(keep, full, mx_ref[...])


def _k1(features_rank, tf_pad, nk_valid):
    q = features_rank.shape[0]
    kpad = tf_pad.shape[0]
    grid = (q // _QT, kpad // _KT)
    return pl.pallas_call(
        functools.partial(_k1_body, nk_valid),
        grid=grid,
        in_specs=[
            pl.BlockSpec((_QT, 128), lambda i, j: (i, 0)),
            pl.BlockSpec((_KT, 128), lambda i, j: (j, 0)),
        ],
        out_specs=[
            pl.BlockSpec((_QT, _KT), lambda i, j: (i, j)),
            pl.BlockSpec((_QT, 128), lambda i, j: (i, j // 4)),
        ],
        out_shape=[
            jax.ShapeDtypeStruct((q, kpad), jnp.float32),
            jax.ShapeDtypeStruct((q, kpad // _CH), jnp.float32),
        ],
        compiler_params=pltpu.CompilerParams(
            dimension_semantics=("parallel", "arbitrary")),
    )(features_rank, tf_pad)


def _k2_body(m_ref, tau_ref):
    b = lax.bitcast_convert_type(m_ref[...], jnp.int32)
    keys = jnp.where(b >= 0, b, (~b) ^ jnp.int32(_INT_MIN))
    n = keys.shape[0]
    lo = jnp.full((n, 1), _INT_MIN, jnp.int32)
    hi = jnp.full((n, 1), jnp.int32(2147483647), jnp.int32)
    for _ in range(32):
        mid = (lo & hi) + ((lo ^ hi) >> 1)
        mid = mid + ((lo ^ hi) & 1)
        cnt = jnp.sum((keys >= mid).astype(jnp.int32), axis=1, keepdims=True)
        ok = cnt >= _MAXK
        lo = jnp.where(ok, mid, lo)
        hi = jnp.where(ok, hi, mid - 1)
    bb = jnp.where(lo >= 0, lo, ~(lo ^ jnp.int32(_INT_MIN)))
    tau_ref[...] = lax.bitcast_convert_type(bb, jnp.float32)


def _k2(mx):
    q, c = mx.shape
    return pl.pallas_call(
        _k2_body,
        grid=(q // _QT,),
        in_specs=[pl.BlockSpec((_QT, c), lambda i: (i, 0))],
        out_specs=pl.BlockSpec((_QT, 1), lambda i: (i, 0)),
        out_shape=jax.ShapeDtypeStruct((q, 1), jnp.float32),
        compiler_params=pltpu.CompilerParams(
            dimension_semantics=("parallel",)),
    )(mx)


def _k3_body(v_ref, l_ref, t_ref, o10_ref, o20_ref, o100_ref):
    v_in = v_ref[...]
    lab = l_ref[...]
    tau = t_ref[...]
    n = v_in.shape[0]
    v0 = jnp.where(v_in >= tau, v_in, -3e38)
    lane = lax.broadcasted_iota(jnp.int32, v0.shape, 1)
    cls = lax.broadcasted_iota(jnp.int32, (n, _NCLS), 1)
    m0 = jnp.max(v0, axis=1, keepdims=True)

    def step(j, carry):
        v, acc, z = carry
        mj = jnp.max(v, axis=1, keepdims=True)
        ej = jnp.exp((mj - m0) / _T)
        pos = jnp.min(jnp.where(v == mj, lane, jnp.int32(2**30)),
                      axis=1, keepdims=True)
        is_sel = lane == pos
        labj = jnp.sum(jnp.where(is_sel, lab, 0), axis=1, keepdims=True)
        acc = acc + jnp.where(cls == labj, ej, 0.0)
        z = z + ej
        v = jnp.where(is_sel, -3e38, v)

        @pl.when(j == 9)
        def _():
            o10_ref[...] = acc

        @pl.when(j == 19)
        def _():
            o20_ref[...] = acc

        return v, acc, z

    _, acc, z = lax.fori_loop(
        0, _MAXK, step,
        (v0, jnp.zeros((n, _NCLS), jnp.float32), jnp.zeros((n, 1), jnp.float32)))
    o100_ref[...] = acc / z
    o10_ref[...] = o10_ref[...] / z
    o20_ref[...] = o20_ref[...] / z


def _k3(cval, clab, tau):
    q, w = cval.shape
    qt = 128
    out_sds = jax.ShapeDtypeStruct((q, _NCLS), jnp.float32)
    return pl.pallas_call(
        _k3_body,
        grid=(q // qt,),
        in_specs=[
            pl.BlockSpec((qt, w), lambda i: (i, 0)),
            pl.BlockSpec((qt, w), lambda i: (i, 0)),
            pl.BlockSpec((qt, 1), lambda i: (i, 0)),
        ],
        out_specs=[
            pl.BlockSpec((qt, _NCLS), lambda i: (i, 0)),
            pl.BlockSpec((qt, _NCLS), lambda i: (i, 0)),
            pl.BlockSpec((qt, _NCLS), lambda i: (i, 0)),
        ],
        out_shape=[out_sds, out_sds, out_sds],
        compiler_params=pltpu.CompilerParams(
            dimension_semantics=("arbitrary",)),
    )(cval, clab, tau)


def kernel(train_features, train_labels, features_rank):
    nk = train_features.shape[0]
    blk = _KT * 4
    kpad = ((nk + blk - 1) // blk) * blk
    nchunks = kpad // _CH
    tf = jnp.pad(train_features, ((0, kpad - nk), (0, 0)))
    sim, mx = _k1(features_rank, tf, nk)
    tau = _k2(mx)
    _, cidx = lax.top_k(mx, _NSEL)
    q = features_rank.shape[0]
    sim3 = sim.reshape(-1, nchunks, _CH)
    cval = jnp.take_along_axis(sim3, cidx[:, :, None], axis=1).reshape(q, -1)
    labrows = jnp.pad(train_labels, (0, kpad - nk)).reshape(nchunks, _CH)
    clab = jnp.take(labrows, cidx, axis=0).reshape(q, -1)
    o10, o20, o100 = _k3(cval.reshape(q, -1), clab.reshape(q, -1), tau)
    return (o10, o20, o100)


# speedup vs baseline: 14.7067x; 1.0177x over previous
import functools

import jax
import jax.numpy as jnp
from jax import lax
from jax.experimental import pallas as pl
from jax.experimental.pallas import tpu as pltpu

_MAXK = 100
_T = 0.07
_NCLS = 1000

_QT = 256
_KT = 2048
_CH = 64
_NSEL = 128
_INT_MIN = -2147483648


def _k1_body(nk_valid, q_ref, t_ref, sim_ref, mx_ref):
    kidx = pl.program_id(1)
    sim = lax.dot_general(q_ref[...], t_ref[...], (((1,), (1,)), ((), ())),
                          preferred_element_type=jnp.float32)
    limit = nk_valid - kidx * _KT
    col = lax.broadcasted_iota(jnp.int32, sim.shape, 1)
    sim = jnp.where(col < limit, sim, -1e30)
    sim_ref[...] = sim
    nch = _KT // _CH
    cmx = jnp.max(sim.reshape(sim.shape[0], nch, _CH), axis=2)
    @pl.when(kidx % 4 == 0)
    def _():
        mx_ref[...] = jnp.full_like(mx_ref[...], -3e38)
    off = (kidx % 4) * nch
    lane_i = lax.broadcasted_iota(jnp.int32, (sim.shape[0], 128), 1)
    full = jnp.concatenate([cmx] * 4, axis=1)
    keep = (lane_i >= off) & (lane_i < off + nch)
    mx_ref[...] = jnp.where(keep, full, mx_ref[...])


def _k1(features_rank, tf_pad, nk_valid):
    q = features_rank.shape[0]
    kpad = tf_pad.shape[0]
    grid = (q // _QT, kpad // _KT)
    return pl.pallas_call(
        functools.partial(_k1_body, nk_valid),
        grid=grid,
        in_specs=[
            pl.BlockSpec((_QT, 128), lambda i, j: (i, 0)),
            pl.BlockSpec((_KT, 128), lambda i, j: (j, 0)),
        ],
        out_specs=[
            pl.BlockSpec((_QT, _KT), lambda i, j: (i, j)),
            pl.BlockSpec((_QT, 128), lambda i, j: (i, j // 4)),
        ],
        out_shape=[
            jax.ShapeDtypeStruct((q, kpad), jnp.float32),
            jax.ShapeDtypeStruct((q, kpad // _CH), jnp.float32),
        ],
        compiler_params=pltpu.CompilerParams(
            dimension_semantics=("parallel", "arbitrary")),
    )(features_rank, tf_pad)


def _k2_body(m_ref, tau_ref):
    b = lax.bitcast_convert_type(m_ref[...], jnp.int32)
    keys = jnp.where(b >= 0, b, (~b) ^ jnp.int32(_INT_MIN))
    n = keys.shape[0]
    lo = jnp.full((n, 1), _INT_MIN, jnp.int32)
    hi = jnp.full((n, 1), jnp.int32(2147483647), jnp.int32)
    for _ in range(32):
        mid = (lo & hi) + ((lo ^ hi) >> 1)
        mid = mid + ((lo ^ hi) & 1)
        cnt = jnp.sum((keys >= mid).astype(jnp.int32), axis=1, keepdims=True)
        ok = cnt >= _MAXK
        lo = jnp.where(ok, mid, lo)
        hi = jnp.where(ok, hi, mid - 1)
    bb = jnp.where(lo >= 0, lo, ~(lo ^ jnp.int32(_INT_MIN)))
    tau_ref[...] = lax.bitcast_convert_type(bb, jnp.float32)


def _k2(mx):
    q, c = mx.shape
    return pl.pallas_call(
        _k2_body,
        grid=(q // _QT,),
        in_specs=[pl.BlockSpec((_QT, c), lambda i: (i, 0))],
        out_specs=pl.BlockSpec((_QT, 1), lambda i: (i, 0)),
        out_shape=jax.ShapeDtypeStruct((q, 1), jnp.float32),
        compiler_params=pltpu.CompilerParams(
            dimension_semantics=("parallel",)),
    )(mx)


def _k3_body(v_ref, l_ref, t_ref, o10_ref, o20_ref, o100_ref):
    v_in = v_ref[...]
    lab = l_ref[...]
    tau = t_ref[...]
    n = v_in.shape[0]
    v0 = jnp.where(v_in >= tau, v_in, -3e38)
    lane = lax.broadcasted_iota(jnp.int32, v0.shape, 1)
    cls = lax.broadcasted_iota(jnp.int32, (n, _NCLS), 1)
    m0 = jnp.max(v0, axis=1, keepdims=True)
    big = jnp.int32(2**30)

    def step(j, carry):
        mcur, lastpos, acc, z = carry
        same_pos = jnp.min(
            jnp.where((v0 == mcur) & (lane > lastpos), lane, big),
            axis=1, keepdims=True)
        has = same_pos < big
        vnext = jnp.max(jnp.where(v0 < mcur, v0, -3e38), axis=1, keepdims=True)
        next_pos = jnp.min(jnp.where(v0 == vnext, lane, big),
                           axis=1, keepdims=True)
        mj = jnp.where(has, mcur, vnext)
        pos = jnp.where(has, same_pos, next_pos)
        ej = jnp.exp((mj - m0) / _T)
        labj = jnp.sum(jnp.where(lane == pos, lab, 0), axis=1, keepdims=True)
        acc = acc + jnp.where(cls == labj, ej, 0.0)
        z = z + ej

        @pl.when(j == 9)
        def _():
            o10_ref[...] = acc

        @pl.when(j == 19)
        def _():
            o20_ref[...] = acc

        return mj, pos, acc, z

    _, _, acc, z = lax.fori_loop(
        0, _MAXK, step,
        (jnp.full((n, 1), 3e38, jnp.float32),
         jnp.full((n, 1), -1, jnp.int32),
         jnp.zeros((n, _NCLS), jnp.float32), jnp.zeros((n, 1), jnp.float32)))
    o100_ref[...] = acc / z
    o10_ref[...] = o10_ref[...] / z
    o20_ref[...] = o20_ref[...] / z


def _k3(cval, clab, tau):
    q, w = cval.shape
    qt = 128
    out_sds = jax.ShapeDtypeStruct((q, _NCLS), jnp.float32)
    return pl.pallas_call(
        _k3_body,
        grid=(q // qt,),
        in_specs=[
            pl.BlockSpec((qt, w), lambda i: (i, 0)),
            pl.BlockSpec((qt, w), lambda i: (i, 0)),
            pl.BlockSpec((qt, 1), lambda i: (i, 0)),
        ],
        out_specs=[
            pl.BlockSpec((qt, _NCLS), lambda i: (i, 0)),
            pl.BlockSpec((qt, _NCLS), lambda i: (i, 0)),
            pl.BlockSpec((qt, _NCLS), lambda i: (i, 0)),
        ],
        out_shape=[out_sds, out_sds, out_sds],
        compiler_params=pltpu.CompilerParams(
            dimension_semantics=("arbitrary",)),
    )(cval, clab, tau)


def kernel(train_features, train_labels, features_rank):
    nk = train_features.shape[0]
    blk = _KT * 4
    kpad = ((nk + blk - 1) // blk) * blk
    nchunks = kpad // _CH
    tf = jnp.pad(train_features, ((0, kpad - nk), (0, 0)))
    sim, mx = _k1(features_rank, tf, nk)
    tau = _k2(mx)
    _, cidx = lax.top_k(mx, _NSEL)
    q = features_rank.shape[0]
    sim3 = sim.reshape(-1, nchunks, _CH)
    cval = jnp.take_along_axis(sim3, cidx[:, :, None], axis=1).reshape(q, -1)
    labrows = jnp.pad(train_labels, (0, kpad - nk)).reshape(nchunks, _CH)
    clab = jnp.take(labrows, cidx, axis=0).reshape(q, -1)
    o10, o20, o100 = _k3(cval.reshape(q, -1), clab.reshape(q, -1), tau)
    return (o10, o20, o100)
